# Initial kernel scaffold; baseline (speedup 1.0000x reference)
import jax
import jax.numpy as jnp
from jax.experimental import pallas as pl


def kernel(invariant_features, positions, cell, W1, b1, W2, b2, Wf1, bf1, Wf2, bf2, Wf3, bf3):
    raise NotImplementedError("write your pallas kernel here")


# speedup vs baseline: 9.6615x; 9.6615x over previous
import functools
import math

import jax
import jax.numpy as jnp
import numpy as np
from jax import lax
from jax.experimental import pallas as pl
from jax.experimental.pallas import tpu as pltpu
from jax.experimental.pallas import tpu_sc as plsc

N = 50000
F = 256
H = 128
C = 4
FH = 64
M = 32
MESH = M * M * M

BLK_A = 1024
GRID_A = (N + BLK_A - 1) // BLK_A
NP = GRID_A * BLK_A
NCHUNK = 8
CHUNK = NP // NCHUNK
G16 = CHUNK // 16

_CORNERS = ((0, 0, 0), (0, 0, 1), (0, 1, 0), (0, 1, 1),
            (1, 0, 0), (1, 0, 1), (1, 1, 0), (1, 1, 1))

_j = np.arange(M)
_Fc = np.exp(-2j * np.pi * np.outer(_j, _j) / M)
_F2c = np.kron(_Fc, _Fc)
F2R = _F2c.real.astype(np.float32)
F2I = _F2c.imag.astype(np.float32)
_BAc = np.kron(np.eye(C), _Fc)
BAR = _BAc.real.astype(np.float32)
BAI = _BAc.imag.astype(np.float32)

_n = np.fft.fftfreq(M) * M
_nx, _ny, _nz = np.meshgrid(_n, _n, _n, indexing="ij")
KINT = np.stack([_nx.ravel(), _ny.ravel(), _nz.ravel()], axis=1).astype(np.float32)


def _stage_a_body(feat_ref, posT_ref, icT_ref, w1_ref, b1_ref, w2_ref, b2_ref,
                  src_ref, idx_ref, w_ref):
    x = feat_ref[...]
    hT = lax.dot_general(w1_ref[...], x, (((0,), (1,)), ((), ())),
                         preferred_element_type=jnp.float32)
    hT = hT + b1_ref[...]
    hT = hT * jax.nn.sigmoid(hT)
    sT = lax.dot_general(w2_ref[...], hT, (((0,), (0,)), ((), ())),
                         preferred_element_type=jnp.float32)
    sT = sT + b2_ref[...]

    pT = posT_ref[...]
    fr = jnp.dot(icT_ref[...], pT, preferred_element_type=jnp.float32)
    fr = fr - jnp.floor(fr)
    sc = fr * float(M)
    base = jnp.floor(sc)
    fo = sc - base
    bi = base.astype(jnp.int32)

    bx, by, bz = bi[0:1, :], bi[1:2, :], bi[2:3, :]
    fx, fy, fz = fo[0:1, :], fo[1:2, :], fo[2:3, :]
    one = jnp.float32(1.0)
    w_rows = []
    i_rows = []
    for (ox, oy, oz) in _CORNERS:
        ix = bx + ox
        iy = by + oy
        iz = bz + oz
        ix = jnp.where(ix >= M, ix - M, ix)
        iy = jnp.where(iy >= M, iy - M, iy)
        iz = jnp.where(iz >= M, iz - M, iz)
        flat = (ix * M + iy) * M + iz
        wx = fx if ox else one - fx
        wy = fy if oy else one - fy
        wz = fz if oz else one - fz
        i_rows.append(flat)
        w_rows.append(wx * wy * wz)
    icat = jnp.concatenate(i_rows, axis=0)
    wcat = jnp.concatenate(w_rows, axis=0)

    gid = pl.program_id(0) * BLK_A + lax.broadcasted_iota(jnp.int32, (1, BLK_A), 1)
    valid = gid < N
    src_ref[...] = jnp.where(valid, sT, 0.0)
    idx_ref[...] = jnp.where(valid, icat, 0)
    w_ref[...] = jnp.where(valid, wcat, 0.0)


def _stage_a(features, posT, inv_cellT, W1, b1c, W2, b2c):
    return pl.pallas_call(
        _stage_a_body,
        grid=(GRID_A,),
        in_specs=[
            pl.BlockSpec((BLK_A, F), lambda n: (n, 0)),
            pl.BlockSpec((3, BLK_A), lambda n: (0, n)),
            pl.BlockSpec((3, 3), lambda n: (0, 0)),
            pl.BlockSpec((F, H), lambda n: (0, 0)),
            pl.BlockSpec((H, 1), lambda n: (0, 0)),
            pl.BlockSpec((H, C), lambda n: (0, 0)),
            pl.BlockSpec((C, 1), lambda n: (0, 0)),
        ],
        out_specs=[
            pl.BlockSpec((C, BLK_A), lambda n: (0, n)),
            pl.BlockSpec((8, BLK_A), lambda n: (0, n)),
            pl.BlockSpec((8, BLK_A), lambda n: (0, n)),
        ],
        out_shape=[
            jax.ShapeDtypeStruct((C, NP), jnp.float32),
            jax.ShapeDtypeStruct((8, NP), jnp.int32),
            jax.ShapeDtypeStruct((8, NP), jnp.float32),
        ],
    )(features, posT, inv_cellT, W1, b1c, W2, b2c)


def _scatter_body(srcT_hbm, idx8_hbm, w8_hbm, out_hbm,
                  meshbuf, idxbuf, wbuf, srcbuf):
    wid = lax.axis_index("s") * 2 + lax.axis_index("c")
    p = wid // C
    c = wid % C
    base = p * CHUNK

    def zero_body(i, _):
        meshbuf[pl.ds(i * 16, 16)] = jnp.zeros((16,), jnp.float32)
        return 0

    lax.fori_loop(0, MESH // 16, zero_body, 0)

    pltpu.sync_copy(srcT_hbm.at[c, pl.ds(base, CHUNK)], srcbuf)
    for j in range(8):
        pltpu.sync_copy(idx8_hbm.at[j, pl.ds(base, CHUNK)], idxbuf)
        pltpu.sync_copy(w8_hbm.at[j, pl.ds(base, CHUNK)], wbuf)

        def scat_body(g, _):
            off = g * 16
            iv = idxbuf[pl.ds(off, 16)]
            vv = srcbuf[pl.ds(off, 16)] * wbuf[pl.ds(off, 16)]
            plsc.addupdate_scatter(meshbuf, [iv], vv)
            return 0

        lax.fori_loop(0, G16, scat_body, 0)

    pltpu.sync_copy(meshbuf, out_hbm.at[p, pl.ds(c * MESH, MESH)])


def _stage_b(srcT, idx8, w8):
    mesh = plsc.VectorSubcoreMesh(core_axis_name="c", subcore_axis_name="s",
                                  num_cores=2, num_subcores=16)
    k = pl.kernel(
        _scatter_body,
        out_type=jax.ShapeDtypeStruct((NCHUNK, C * MESH), jnp.float32),
        mesh=mesh,
        scratch_types=[
            pltpu.VMEM((MESH,), jnp.float32),
            pltpu.VMEM((CHUNK,), jnp.int32),
            pltpu.VMEM((CHUNK,), jnp.float32),
            pltpu.VMEM((CHUNK,), jnp.float32),
        ],
        compiler_params=pltpu.CompilerParams(
            needs_layout_passes=False, use_tc_tiling_on_sc=False),
    )
    return k(srcT, idx8, w8)


def _filter_body(kint_ref, ict_ref, vol_ref,
                 wf1_ref, bf1_ref, wf2_ref, bf2_ref, wf3_ref, bf3_ref,
                 out_ref):
    twopi = jnp.float32(2.0 * math.pi)
    kv = jnp.dot(kint_ref[...], ict_ref[...],
                 preferred_element_type=jnp.float32) * twopi
    k2 = jnp.sum(kv * kv, axis=1, keepdims=True)
    knorm = jnp.sqrt(k2)
    safe_k = jnp.maximum(knorm, 1e-6)
    x0 = jnp.log1p(safe_k)
    x1 = x0 * x0
    x2 = 1.0 / safe_k
    xf = jnp.concatenate([x0, x1, x2], axis=1)
    h1 = jnp.dot(xf, wf1_ref[...], preferred_element_type=jnp.float32) + bf1_ref[...]
    h1 = h1 * jax.nn.sigmoid(h1)
    h2 = jnp.dot(h1, wf2_ref[...], preferred_element_type=jnp.float32) + bf2_ref[...]
    h2 = h2 * jax.nn.sigmoid(h2)
    z = jnp.dot(h2, wf3_ref[...], preferred_element_type=jnp.float32) + bf3_ref[...]
    scale = jnp.maximum(z, 0.0) + jnp.log1p(jnp.exp(-jnp.abs(z)))
    coef = jnp.float32(4.0 * math.pi) * (jnp.float32(0.5) / vol_ref[0, 0])
    kern = coef / (safe_k * safe_k) * scale
    row = pl.program_id(0) * BLK_K + lax.broadcasted_iota(jnp.int32, (BLK_K, 1), 0)
    out_ref[...] = jnp.where(row == 0, 0.0, kern)


BLK_K = 4096


def _stage_c2(inv_cellT, volarr, Wf1, bf1r, Wf2, bf2r, Wf3, bf3r):
    return pl.pallas_call(
        _filter_body,
        grid=(MESH // BLK_K,),
        in_specs=[
            pl.BlockSpec((BLK_K, 3), lambda n: (n, 0)),
            pl.BlockSpec((3, 3), lambda n: (0, 0)),
            pl.BlockSpec(memory_space=pltpu.SMEM),
            pl.BlockSpec((3, FH), lambda n: (0, 0)),
            pl.BlockSpec((1, FH), lambda n: (0, 0)),
            pl.BlockSpec((FH, FH), lambda n: (0, 0)),
            pl.BlockSpec((1, FH), lambda n: (0, 0)),
            pl.BlockSpec((FH, 1), lambda n: (0, 0)),
            pl.BlockSpec((1, 1), lambda n: (0, 0)),
        ],
        out_specs=pl.BlockSpec((BLK_K, 1), lambda n: (n, 0)),
        out_shape=jax.ShapeDtypeStruct((MESH, 1), jnp.float32),
    )(KINT, inv_cellT, volarr, Wf1, bf1r, Wf2, bf2r, Wf3, bf3r)


def _dft_body(p_ref, kern_ref, f2r_ref, f2i_ref, bar_ref, bai_ref, phi_ref):
    X = jnp.sum(p_ref[...], axis=0)
    f2r = f2r_ref[...]
    f2i = f2i_ref[...]
    bar = bar_ref[...]
    bai = bai_ref[...]
    dot = functools.partial(jnp.dot, preferred_element_type=jnp.float32)
    r1 = dot(X, f2r)
    i1 = dot(X, f2i)
    r2 = dot(bar, r1) - dot(bai, i1)
    i2 = dot(bar, i1) + dot(bai, r1)
    k2 = kern_ref[...]
    kt = jnp.concatenate([k2, k2, k2, k2], axis=0)
    r3 = r2 * kt
    i3 = i2 * kt
    inv1 = jnp.float32(1.0 / (M * M))
    r4 = (dot(r3, f2r) + dot(i3, f2i)) * inv1
    i4 = (dot(i3, f2r) - dot(r3, f2i)) * inv1
    phi_ref[...] = (dot(bar, r4) + dot(bai, i4)) * jnp.float32(1.0 / M)


def _stage_c(partials, kern2):
    return pl.pallas_call(
        _dft_body,
        in_specs=[
            pl.BlockSpec((NCHUNK, C * M, M * M), lambda: (0, 0, 0)),
            pl.BlockSpec((M, M * M), lambda: (0, 0)),
            pl.BlockSpec((M * M, M * M), lambda: (0, 0)),
            pl.BlockSpec((M * M, M * M), lambda: (0, 0)),
            pl.BlockSpec((C * M, C * M), lambda: (0, 0)),
            pl.BlockSpec((C * M, C * M), lambda: (0, 0)),
        ],
        out_specs=pl.BlockSpec((C * M, M * M), lambda: (0, 0)),
        out_shape=jax.ShapeDtypeStruct((C * M, M * M), jnp.float32),
    )(partials, kern2, F2R, F2I, BAR, BAI)


def _gather_body(phi_hbm, srcT_hbm, idx8_hbm, w8_hbm, out_hbm,
                 phibuf, potbuf, idxbuf, wbuf, srcbuf):
    wid = lax.axis_index("s") * 2 + lax.axis_index("c")
    p = wid // C
    c = wid % C
    base = p * CHUNK

    pltpu.sync_copy(phi_hbm.at[c], phibuf)

    def zero_body(g, _):
        potbuf[pl.ds(g * 16, 16)] = jnp.zeros((16,), jnp.float32)
        return 0

    lax.fori_loop(0, G16, zero_body, 0)

    for j in range(8):
        pltpu.sync_copy(idx8_hbm.at[j, pl.ds(base, CHUNK)], idxbuf)
        pltpu.sync_copy(w8_hbm.at[j, pl.ds(base, CHUNK)], wbuf)

        def gat_body(g, _):
            off = g * 16
            iv = idxbuf[pl.ds(off, 16)]
            vals = plsc.load_gather(phibuf, [iv])
            potbuf[pl.ds(off, 16)] = potbuf[pl.ds(off, 16)] + vals * wbuf[pl.ds(off, 16)]
            return 0

        lax.fori_loop(0, G16, gat_body, 0)

    pltpu.sync_copy(srcT_hbm.at[c, pl.ds(base, CHUNK)], srcbuf)

    def mul_body(g, _):
        off = g * 16
        potbuf[pl.ds(off, 16)] = potbuf[pl.ds(off, 16)] * srcbuf[pl.ds(off, 16)]
        return 0

    lax.fori_loop(0, G16, mul_body, 0)
    pltpu.sync_copy(potbuf, out_hbm.at[c, pl.ds(base, CHUNK)])


def _stage_d(phi, srcT, idx8, w8):
    mesh = plsc.VectorSubcoreMesh(core_axis_name="c", subcore_axis_name="s",
                                  num_cores=2, num_subcores=16)
    k = pl.kernel(
        _gather_body,
        out_type=jax.ShapeDtypeStruct((C, NP), jnp.float32),
        mesh=mesh,
        scratch_types=[
            pltpu.VMEM((MESH,), jnp.float32),
            pltpu.VMEM((CHUNK,), jnp.float32),
            pltpu.VMEM((CHUNK,), jnp.int32),
            pltpu.VMEM((CHUNK,), jnp.float32),
            pltpu.VMEM((CHUNK,), jnp.float32),
        ],
        compiler_params=pltpu.CompilerParams(
            needs_layout_passes=False, use_tc_tiling_on_sc=False),
    )
    return k(phi, srcT, idx8, w8)


def _reduce_body(e_ref, out_ref):
    out_ref[...] = jnp.sum(e_ref[...], axis=0)


def _stage_e(epart):
    blk = 512
    return pl.pallas_call(
        _reduce_body,
        grid=(NP // blk,),
        in_specs=[pl.BlockSpec((C, blk), lambda n: (0, n))],
        out_specs=pl.BlockSpec((blk,), lambda n: (n,)),
        out_shape=jax.ShapeDtypeStruct((N,), jnp.float32),
    )(epart)


def kernel(invariant_features, positions, cell, W1, b1, W2, b2,
           Wf1, bf1, Wf2, bf2, Wf3, bf3):
    inv_cell = jnp.linalg.inv(cell)
    inv_cellT = inv_cell.T
    vol = jnp.abs(jnp.linalg.det(cell))
    volarr = vol.reshape(1, 1)
    posT = positions.T
    b1c = b1.reshape(H, 1)
    b2c = b2.reshape(C, 1)
    bf1r = bf1.reshape(1, FH)
    bf2r = bf2.reshape(1, FH)
    bf3r = bf3.reshape(1, 1)

    srcT, idx8, w8 = _stage_a(invariant_features, posT, inv_cellT, W1, b1c, W2, b2c)
    partials = _stage_b(srcT, idx8, w8)
    kern_col = _stage_c2(inv_cellT, volarr, Wf1, bf1r, Wf2, bf2r, Wf3, bf3r)
    kern2 = kern_col.reshape(M, M * M)
    phi = _stage_c(partials.reshape(NCHUNK, C * M, M * M), kern2)
    epart = _stage_d(phi.reshape(C, MESH), srcT, idx8, w8)
    return _stage_e(epart)


# speedup vs baseline: 10.3303x; 1.0692x over previous
import functools
import math

import jax
import jax.numpy as jnp
import numpy as np
from jax import lax
from jax.experimental import pallas as pl
from jax.experimental.pallas import tpu as pltpu
from jax.experimental.pallas import tpu_sc as plsc

N = 50000
F = 256
H = 128
C = 4
FH = 64
M = 32
MESH = M * M * M

BLK_A = 1024
GRID_A = (N + BLK_A - 1) // BLK_A
NP = GRID_A * BLK_A
NCHUNK = 8
CHUNK = NP // NCHUNK
G16 = CHUNK // 16

_CORNERS = ((0, 0, 0), (0, 0, 1), (0, 1, 0), (0, 1, 1),
            (1, 0, 0), (1, 0, 1), (1, 1, 0), (1, 1, 1))

_j = np.arange(M)
_Fc = np.exp(-2j * np.pi * np.outer(_j, _j) / M)
_F2c = np.kron(_Fc, _Fc)
F2R = _F2c.real.astype(np.float32)
F2I = _F2c.imag.astype(np.float32)
_BAc = np.kron(np.eye(C), _Fc)
BAR = _BAc.real.astype(np.float32)
BAI = _BAc.imag.astype(np.float32)

_n = np.fft.fftfreq(M) * M
_nx, _ny, _nz = np.meshgrid(_n, _n, _n, indexing="ij")
KINT = np.stack([_nx.ravel(), _ny.ravel(), _nz.ravel()], axis=1).astype(np.float32)


def _stage_a_body(feat_ref, posT_ref, icT_ref, w1_ref, b1_ref, w2_ref, b2_ref,
                  src_ref, idx_ref, w_ref):
    x = feat_ref[...]
    hT = lax.dot_general(w1_ref[...], x, (((0,), (1,)), ((), ())),
                         preferred_element_type=jnp.float32)
    hT = hT + b1_ref[...]
    hT = hT * jax.nn.sigmoid(hT)
    sT = lax.dot_general(w2_ref[...], hT, (((0,), (0,)), ((), ())),
                         preferred_element_type=jnp.float32)
    sT = sT + b2_ref[...]

    pT = posT_ref[...]
    fr = jnp.dot(icT_ref[...], pT, preferred_element_type=jnp.float32)
    fr = fr - jnp.floor(fr)
    sc = fr * float(M)
    base = jnp.floor(sc)
    fo = sc - base
    bi = base.astype(jnp.int32)

    bx, by, bz = bi[0:1, :], bi[1:2, :], bi[2:3, :]
    fx, fy, fz = fo[0:1, :], fo[1:2, :], fo[2:3, :]
    one = jnp.float32(1.0)
    w_rows = []
    i_rows = []
    for (ox, oy, oz) in _CORNERS:
        ix = bx + ox
        iy = by + oy
        iz = bz + oz
        ix = jnp.where(ix >= M, ix - M, ix)
        iy = jnp.where(iy >= M, iy - M, iy)
        iz = jnp.where(iz >= M, iz - M, iz)
        flat = (ix * M + iy) * M + iz
        wx = fx if ox else one - fx
        wy = fy if oy else one - fy
        wz = fz if oz else one - fz
        i_rows.append(flat)
        w_rows.append(wx * wy * wz)
    icat = jnp.concatenate(i_rows, axis=0)
    wcat = jnp.concatenate(w_rows, axis=0)

    gid = pl.program_id(0) * BLK_A + lax.broadcasted_iota(jnp.int32, (1, BLK_A), 1)
    valid = gid < N
    src_ref[...] = jnp.where(valid, sT, 0.0)
    idx_ref[...] = jnp.where(valid, icat, 0)
    w_ref[...] = jnp.where(valid, wcat, 0.0)


def _stage_a(features, posT, inv_cellT, W1, b1c, W2, b2c):
    return pl.pallas_call(
        _stage_a_body,
        grid=(GRID_A,),
        in_specs=[
            pl.BlockSpec((BLK_A, F), lambda n: (n, 0)),
            pl.BlockSpec((3, BLK_A), lambda n: (0, n)),
            pl.BlockSpec((3, 3), lambda n: (0, 0)),
            pl.BlockSpec((F, H), lambda n: (0, 0)),
            pl.BlockSpec((H, 1), lambda n: (0, 0)),
            pl.BlockSpec((H, C), lambda n: (0, 0)),
            pl.BlockSpec((C, 1), lambda n: (0, 0)),
        ],
        out_specs=[
            pl.BlockSpec((C, BLK_A), lambda n: (0, n)),
            pl.BlockSpec((8, BLK_A), lambda n: (0, n)),
            pl.BlockSpec((8, BLK_A), lambda n: (0, n)),
        ],
        out_shape=[
            jax.ShapeDtypeStruct((C, NP), jnp.float32),
            jax.ShapeDtypeStruct((8, NP), jnp.int32),
            jax.ShapeDtypeStruct((8, NP), jnp.float32),
        ],
    )(features, posT, inv_cellT, W1, b1c, W2, b2c)


def _scatter_body(srcT_hbm, idx8_hbm, w8_hbm, out_hbm,
                  meshbuf, ibuf0, ibuf1, wbuf0, wbuf1, srcbuf,
                  sem0, sem1, sems):
    wid = lax.axis_index("s") * 2 + lax.axis_index("c")
    p = wid // C
    c = wid % C
    base = p * CHUNK
    ibufs = (ibuf0, ibuf1)
    wbufs = (wbuf0, wbuf1)
    semsp = (sem0, sem1)
    cps = [None, None]

    def start(j):
        par = j & 1
        c1 = pltpu.make_async_copy(idx8_hbm.at[j, pl.ds(base, CHUNK)],
                                   ibufs[par], semsp[par])
        c2 = pltpu.make_async_copy(w8_hbm.at[j, pl.ds(base, CHUNK)],
                                   wbufs[par], semsp[par])
        c1.start()
        c2.start()
        cps[par] = (c1, c2)

    cp_src = pltpu.make_async_copy(srcT_hbm.at[c, pl.ds(base, CHUNK)],
                                   srcbuf, sems)
    cp_src.start()
    start(0)

    def zero_body(i, _):
        meshbuf[pl.ds(i * 16, 16)] = jnp.zeros((16,), jnp.float32)
        return 0

    lax.fori_loop(0, MESH // 16, zero_body, 0)
    cp_src.wait()

    for j in range(8):
        if j < 7:
            start(j + 1)
        par = j & 1
        ib = ibufs[par]
        wb = wbufs[par]
        for cp in cps[par]:
            cp.wait()

        def scat_body(g, _):
            off = g * 16
            iv = ib[pl.ds(off, 16)]
            vv = srcbuf[pl.ds(off, 16)] * wb[pl.ds(off, 16)]
            plsc.addupdate_scatter(meshbuf, [iv], vv)
            return 0

        lax.fori_loop(0, G16, scat_body, 0)

    pltpu.sync_copy(meshbuf, out_hbm.at[p, pl.ds(c * MESH, MESH)])


def _stage_b(srcT, idx8, w8):
    mesh = plsc.VectorSubcoreMesh(core_axis_name="c", subcore_axis_name="s",
                                  num_cores=2, num_subcores=16)
    k = pl.kernel(
        _scatter_body,
        out_type=jax.ShapeDtypeStruct((NCHUNK, C * MESH), jnp.float32),
        mesh=mesh,
        scratch_types=[
            pltpu.VMEM((MESH,), jnp.float32),
            pltpu.VMEM((CHUNK,), jnp.int32),
            pltpu.VMEM((CHUNK,), jnp.int32),
            pltpu.VMEM((CHUNK,), jnp.float32),
            pltpu.VMEM((CHUNK,), jnp.float32),
            pltpu.VMEM((CHUNK,), jnp.float32),
            pltpu.SemaphoreType.DMA,
            pltpu.SemaphoreType.DMA,
            pltpu.SemaphoreType.DMA,
        ],
        compiler_params=pltpu.CompilerParams(
            needs_layout_passes=False, use_tc_tiling_on_sc=False),
    )
    return k(srcT, idx8, w8)


def _filter_body(kint_ref, ict_ref, vol_ref,
                 wf1_ref, bf1_ref, wf2_ref, bf2_ref, wf3_ref, bf3_ref,
                 out_ref):
    twopi = jnp.float32(2.0 * math.pi)
    kv = jnp.dot(kint_ref[...], ict_ref[...],
                 preferred_element_type=jnp.float32) * twopi
    k2 = jnp.sum(kv * kv, axis=1, keepdims=True)
    knorm = jnp.sqrt(k2)
    safe_k = jnp.maximum(knorm, 1e-6)
    x0 = jnp.log1p(safe_k)
    x1 = x0 * x0
    x2 = 1.0 / safe_k
    xf = jnp.concatenate([x0, x1, x2], axis=1)
    h1 = jnp.dot(xf, wf1_ref[...], preferred_element_type=jnp.float32) + bf1_ref[...]
    h1 = h1 * jax.nn.sigmoid(h1)
    h2 = jnp.dot(h1, wf2_ref[...], preferred_element_type=jnp.float32) + bf2_ref[...]
    h2 = h2 * jax.nn.sigmoid(h2)
    z = jnp.dot(h2, wf3_ref[...], preferred_element_type=jnp.float32) + bf3_ref[...]
    scale = jnp.maximum(z, 0.0) + jnp.log1p(jnp.exp(-jnp.abs(z)))
    coef = jnp.float32(4.0 * math.pi) * (jnp.float32(0.5) / vol_ref[0, 0])
    kern = coef / (safe_k * safe_k) * scale
    row = pl.program_id(0) * BLK_K + lax.broadcasted_iota(jnp.int32, (BLK_K, 1), 0)
    out_ref[...] = jnp.where(row == 0, 0.0, kern)


BLK_K = 4096


def _stage_c2(inv_cellT, volarr, Wf1, bf1r, Wf2, bf2r, Wf3, bf3r):
    return pl.pallas_call(
        _filter_body,
        grid=(MESH // BLK_K,),
        in_specs=[
            pl.BlockSpec((BLK_K, 3), lambda n: (n, 0)),
            pl.BlockSpec((3, 3), lambda n: (0, 0)),
            pl.BlockSpec(memory_space=pltpu.SMEM),
            pl.BlockSpec((3, FH), lambda n: (0, 0)),
            pl.BlockSpec((1, FH), lambda n: (0, 0)),
            pl.BlockSpec((FH, FH), lambda n: (0, 0)),
            pl.BlockSpec((1, FH), lambda n: (0, 0)),
            pl.BlockSpec((FH, 1), lambda n: (0, 0)),
            pl.BlockSpec((1, 1), lambda n: (0, 0)),
        ],
        out_specs=pl.BlockSpec((BLK_K, 1), lambda n: (n, 0)),
        out_shape=jax.ShapeDtypeStruct((MESH, 1), jnp.float32),
    )(KINT, inv_cellT, volarr, Wf1, bf1r, Wf2, bf2r, Wf3, bf3r)


def _dft_body(p_ref, kern_ref, f2r_ref, f2i_ref, bar_ref, bai_ref, phi_ref):
    X = jnp.sum(p_ref[...], axis=0)
    f2r = f2r_ref[...]
    f2i = f2i_ref[...]
    bar = bar_ref[...]
    bai = bai_ref[...]
    dot = functools.partial(jnp.dot, preferred_element_type=jnp.float32)
    r1 = dot(X, f2r)
    i1 = dot(X, f2i)
    r2 = dot(bar, r1) - dot(bai, i1)
    i2 = dot(bar, i1) + dot(bai, r1)
    k2 = kern_ref[...]
    kt = jnp.concatenate([k2, k2, k2, k2], axis=0)
    r3 = r2 * kt
    i3 = i2 * kt
    inv1 = jnp.float32(1.0 / (M * M))
    r4 = (dot(r3, f2r) + dot(i3, f2i)) * inv1
    i4 = (dot(i3, f2r) - dot(r3, f2i)) * inv1
    phi_ref[...] = (dot(bar, r4) + dot(bai, i4)) * jnp.float32(1.0 / M)


def _stage_c(partials, kern2):
    return pl.pallas_call(
        _dft_body,
        in_specs=[
            pl.BlockSpec((NCHUNK, C * M, M * M), lambda: (0, 0, 0)),
            pl.BlockSpec((M, M * M), lambda: (0, 0)),
            pl.BlockSpec((M * M, M * M), lambda: (0, 0)),
            pl.BlockSpec((M * M, M * M), lambda: (0, 0)),
            pl.BlockSpec((C * M, C * M), lambda: (0, 0)),
            pl.BlockSpec((C * M, C * M), lambda: (0, 0)),
        ],
        out_specs=pl.BlockSpec((C * M, M * M), lambda: (0, 0)),
        out_shape=jax.ShapeDtypeStruct((C * M, M * M), jnp.float32),
    )(partials, kern2, F2R, F2I, BAR, BAI)


def _gather_body(phi_hbm, srcT_hbm, idx8_hbm, w8_hbm, out_hbm,
                 phibuf, potbuf, ibuf0, ibuf1, wbuf0, wbuf1, srcbuf,
                 sem0, sem1, sems):
    wid = lax.axis_index("s") * 2 + lax.axis_index("c")
    p = wid // C
    c = wid % C
    base = p * CHUNK
    ibufs = (ibuf0, ibuf1)
    wbufs = (wbuf0, wbuf1)
    semsp = (sem0, sem1)
    cps = [None, None]

    def start(j):
        par = j & 1
        c1 = pltpu.make_async_copy(idx8_hbm.at[j, pl.ds(base, CHUNK)],
                                   ibufs[par], semsp[par])
        c2 = pltpu.make_async_copy(w8_hbm.at[j, pl.ds(base, CHUNK)],
                                   wbufs[par], semsp[par])
        c1.start()
        c2.start()
        cps[par] = (c1, c2)

    cp_phi = pltpu.make_async_copy(phi_hbm.at[c], phibuf, sems)
    cp_phi.start()
    cp_src = pltpu.make_async_copy(srcT_hbm.at[c, pl.ds(base, CHUNK)],
                                   srcbuf, sems)
    cp_src.start()
    start(0)

    def zero_body(g, _):
        potbuf[pl.ds(g * 16, 16)] = jnp.zeros((16,), jnp.float32)
        return 0

    lax.fori_loop(0, G16, zero_body, 0)
    cp_phi.wait()
    cp_src.wait()

    for j in range(8):
        if j < 7:
            start(j + 1)
        par = j & 1
        ib = ibufs[par]
        wb = wbufs[par]
        for cp in cps[par]:
            cp.wait()

        def gat_body(g, _):
            off = g * 16
            iv = ib[pl.ds(off, 16)]
            vals = plsc.load_gather(phibuf, [iv])
            potbuf[pl.ds(off, 16)] = potbuf[pl.ds(off, 16)] + vals * wb[pl.ds(off, 16)]
            return 0

        lax.fori_loop(0, G16, gat_body, 0)

    def mul_body(g, _):
        off = g * 16
        potbuf[pl.ds(off, 16)] = potbuf[pl.ds(off, 16)] * srcbuf[pl.ds(off, 16)]
        return 0

    lax.fori_loop(0, G16, mul_body, 0)
    pltpu.sync_copy(potbuf, out_hbm.at[c, pl.ds(base, CHUNK)])


def _stage_d(phi, srcT, idx8, w8):
    mesh = plsc.VectorSubcoreMesh(core_axis_name="c", subcore_axis_name="s",
                                  num_cores=2, num_subcores=16)
    k = pl.kernel(
        _gather_body,
        out_type=jax.ShapeDtypeStruct((C, NP), jnp.float32),
        mesh=mesh,
        scratch_types=[
            pltpu.VMEM((MESH,), jnp.float32),
            pltpu.VMEM((CHUNK,), jnp.float32),
            pltpu.VMEM((CHUNK,), jnp.int32),
            pltpu.VMEM((CHUNK,), jnp.int32),
            pltpu.VMEM((CHUNK,), jnp.float32),
            pltpu.VMEM((CHUNK,), jnp.float32),
            pltpu.VMEM((CHUNK,), jnp.float32),
            pltpu.SemaphoreType.DMA,
            pltpu.SemaphoreType.DMA,
            pltpu.SemaphoreType.DMA,
        ],
        compiler_params=pltpu.CompilerParams(
            needs_layout_passes=False, use_tc_tiling_on_sc=False),
    )
    return k(phi, srcT, idx8, w8)


def _reduce_body(e_ref, out_ref):
    out_ref[...] = jnp.sum(e_ref[...], axis=0)


def _stage_e(epart):
    blk = 512
    return pl.pallas_call(
        _reduce_body,
        grid=(NP // blk,),
        in_specs=[pl.BlockSpec((C, blk), lambda n: (0, n))],
        out_specs=pl.BlockSpec((blk,), lambda n: (n,)),
        out_shape=jax.ShapeDtypeStruct((N,), jnp.float32),
    )(epart)


def kernel(invariant_features, positions, cell, W1, b1, W2, b2,
           Wf1, bf1, Wf2, bf2, Wf3, bf3):
    inv_cell = jnp.linalg.inv(cell)
    inv_cellT = inv_cell.T
    vol = jnp.abs(jnp.linalg.det(cell))
    volarr = vol.reshape(1, 1)
    posT = positions.T
    b1c = b1.reshape(H, 1)
    b2c = b2.reshape(C, 1)
    bf1r = bf1.reshape(1, FH)
    bf2r = bf2.reshape(1, FH)
    bf3r = bf3.reshape(1, 1)

    srcT, idx8, w8 = _stage_a(invariant_features, posT, inv_cellT, W1, b1c, W2, b2c)
    partials = _stage_b(srcT, idx8, w8)
    kern_col = _stage_c2(inv_cellT, volarr, Wf1, bf1r, Wf2, bf2r, Wf3, bf3r)
    kern2 = kern_col.reshape(M, M * M)
    phi = _stage_c(partials.reshape(NCHUNK, C * M, M * M), kern2)
    epart = _stage_d(phi.reshape(C, MESH), srcT, idx8, w8)
    return _stage_e(epart)


# speedup vs baseline: 12.6559x; 1.2251x over previous
import functools
import math

import jax
import jax.numpy as jnp
import numpy as np
from jax import lax
from jax.experimental import pallas as pl
from jax.experimental.pallas import tpu as pltpu
from jax.experimental.pallas import tpu_sc as plsc

N = 50000
F = 256
H = 128
C = 4
FH = 64
M = 32
MESH = M * M * M

BLK_A = 1024
GRID_A = (N + BLK_A - 1) // BLK_A
NP = GRID_A * BLK_A
NCHUNK = 8
CHUNK = NP // NCHUNK
G16 = CHUNK // 16

_CORNERS = ((0, 0, 0), (0, 0, 1), (0, 1, 0), (0, 1, 1),
            (1, 0, 0), (1, 0, 1), (1, 1, 0), (1, 1, 1))

_j = np.arange(M)
_Fc = np.exp(-2j * np.pi * np.outer(_j, _j) / M)
_F2c = np.kron(_Fc, _Fc)
F2R = _F2c.real.astype(np.float32)
F2I = _F2c.imag.astype(np.float32)
_BAc = np.kron(np.eye(C), _Fc)
BAR = _BAc.real.astype(np.float32)
BAI = _BAc.imag.astype(np.float32)

_n = np.fft.fftfreq(M) * M
_nx, _ny, _nz = np.meshgrid(_n, _n, _n, indexing="ij")
KINT = np.stack([_nx.ravel(), _ny.ravel(), _nz.ravel()], axis=1).astype(np.float32)


def _stage_a_body(feat_ref, posT_ref, icT_ref, w1_ref, b1_ref, w2_ref, b2_ref,
                  src_ref, idx_ref, w_ref):
    x = feat_ref[...]
    hT = lax.dot_general(w1_ref[...], x, (((0,), (1,)), ((), ())),
                         preferred_element_type=jnp.float32)
    hT = hT + b1_ref[...]
    hT = hT * jax.nn.sigmoid(hT)
    sT = lax.dot_general(w2_ref[...], hT, (((0,), (0,)), ((), ())),
                         preferred_element_type=jnp.float32)
    sT = sT + b2_ref[...]

    pT = posT_ref[...]
    fr = jnp.dot(icT_ref[...], pT, preferred_element_type=jnp.float32)
    fr = fr - jnp.floor(fr)
    sc = fr * float(M)
    base = jnp.floor(sc)
    fo = sc - base
    bi = base.astype(jnp.int32)

    bx, by, bz = bi[0:1, :], bi[1:2, :], bi[2:3, :]
    fx, fy, fz = fo[0:1, :], fo[1:2, :], fo[2:3, :]
    one = jnp.float32(1.0)
    w_rows = []
    i_rows = []
    for (ox, oy, oz) in _CORNERS:
        ix = bx + ox
        iy = by + oy
        iz = bz + oz
        ix = jnp.where(ix >= M, ix - M, ix)
        iy = jnp.where(iy >= M, iy - M, iy)
        iz = jnp.where(iz >= M, iz - M, iz)
        flat = (ix * M + iy) * M + iz
        wx = fx if ox else one - fx
        wy = fy if oy else one - fy
        wz = fz if oz else one - fz
        i_rows.append(flat)
        w_rows.append(wx * wy * wz)
    icat = jnp.concatenate(i_rows, axis=0)
    wcat = jnp.concatenate(w_rows, axis=0)

    gid = pl.program_id(0) * BLK_A + lax.broadcasted_iota(jnp.int32, (1, BLK_A), 1)
    valid = gid < N
    src_ref[...] = jnp.where(valid, sT, 0.0)
    idx_ref[...] = jnp.where(valid, icat, 0)
    w_ref[...] = jnp.where(valid, wcat, 0.0)


def _stage_a(features, posT, inv_cellT, W1, b1c, W2, b2c):
    return pl.pallas_call(
        _stage_a_body,
        grid=(GRID_A,),
        in_specs=[
            pl.BlockSpec((BLK_A, F), lambda n: (n, 0)),
            pl.BlockSpec((3, BLK_A), lambda n: (0, n)),
            pl.BlockSpec((3, 3), lambda n: (0, 0)),
            pl.BlockSpec((F, H), lambda n: (0, 0)),
            pl.BlockSpec((H, 1), lambda n: (0, 0)),
            pl.BlockSpec((H, C), lambda n: (0, 0)),
            pl.BlockSpec((C, 1), lambda n: (0, 0)),
        ],
        out_specs=[
            pl.BlockSpec((C, BLK_A), lambda n: (0, n)),
            pl.BlockSpec((8, BLK_A), lambda n: (0, n)),
            pl.BlockSpec((8, BLK_A), lambda n: (0, n)),
        ],
        out_shape=[
            jax.ShapeDtypeStruct((C, NP), jnp.float32),
            jax.ShapeDtypeStruct((8, NP), jnp.int32),
            jax.ShapeDtypeStruct((8, NP), jnp.float32),
        ],
    )(features, posT, inv_cellT, W1, b1c, W2, b2c)


def _scatter_body(srcT_hbm, idx8_hbm, w8_hbm, out_hbm,
                  meshbuf, ibuf0, ibuf1, wbuf0, wbuf1, srcbuf,
                  sem0, sem1, sems):
    wid = lax.axis_index("s") * 2 + lax.axis_index("c")
    p = wid // C
    c = wid % C
    base = p * CHUNK
    ibufs = (ibuf0, ibuf1)
    wbufs = (wbuf0, wbuf1)
    semsp = (sem0, sem1)
    cps = [None, None]

    def start(j):
        par = j & 1
        c1 = pltpu.make_async_copy(idx8_hbm.at[j, pl.ds(base, CHUNK)],
                                   ibufs[par], semsp[par])
        c2 = pltpu.make_async_copy(w8_hbm.at[j, pl.ds(base, CHUNK)],
                                   wbufs[par], semsp[par])
        c1.start()
        c2.start()
        cps[par] = (c1, c2)

    cp_src = pltpu.make_async_copy(srcT_hbm.at[c, pl.ds(base, CHUNK)],
                                   srcbuf, sems)
    cp_src.start()
    start(0)

    def zero_body(i, _):
        for u in range(8):
            meshbuf[pl.ds(i * 128 + u * 16, 16)] = jnp.zeros((16,), jnp.float32)
        return 0

    lax.fori_loop(0, MESH // 128, zero_body, 0)
    cp_src.wait()

    for j in range(8):
        if j < 7:
            start(j + 1)
        par = j & 1
        ib = ibufs[par]
        wb = wbufs[par]
        for cp in cps[par]:
            cp.wait()

        def scat_body(g, _):
            for u in range(4):
                off = g * 64 + u * 16
                iv = ib[pl.ds(off, 16)]
                vv = srcbuf[pl.ds(off, 16)] * wb[pl.ds(off, 16)]
                plsc.addupdate_scatter(meshbuf, [iv], vv)
            return 0

        lax.fori_loop(0, G16 // 4, scat_body, 0)

    pltpu.sync_copy(meshbuf, out_hbm.at[p, pl.ds(c * MESH, MESH)])


def _stage_b(srcT, idx8, w8):
    mesh = plsc.VectorSubcoreMesh(core_axis_name="c", subcore_axis_name="s",
                                  num_cores=2, num_subcores=16)
    k = pl.kernel(
        _scatter_body,
        out_type=jax.ShapeDtypeStruct((NCHUNK, C * MESH), jnp.float32),
        mesh=mesh,
        scratch_types=[
            pltpu.VMEM((MESH,), jnp.float32),
            pltpu.VMEM((CHUNK,), jnp.int32),
            pltpu.VMEM((CHUNK,), jnp.int32),
            pltpu.VMEM((CHUNK,), jnp.float32),
            pltpu.VMEM((CHUNK,), jnp.float32),
            pltpu.VMEM((CHUNK,), jnp.float32),
            pltpu.SemaphoreType.DMA,
            pltpu.SemaphoreType.DMA,
            pltpu.SemaphoreType.DMA,
        ],
        compiler_params=pltpu.CompilerParams(
            needs_layout_passes=False, use_tc_tiling_on_sc=False),
    )
    return k(srcT, idx8, w8)


def _filter_body(kint_ref, ict_ref, vol_ref,
                 wf1_ref, bf1_ref, wf2_ref, bf2_ref, wf3_ref, bf3_ref,
                 out_ref):
    twopi = jnp.float32(2.0 * math.pi)
    kv = jnp.dot(kint_ref[...], ict_ref[...],
                 preferred_element_type=jnp.float32) * twopi
    k2 = jnp.sum(kv * kv, axis=1, keepdims=True)
    knorm = jnp.sqrt(k2)
    safe_k = jnp.maximum(knorm, 1e-6)
    x0 = jnp.log1p(safe_k)
    x1 = x0 * x0
    x2 = 1.0 / safe_k
    xf = jnp.concatenate([x0, x1, x2], axis=1)
    h1 = jnp.dot(xf, wf1_ref[...], preferred_element_type=jnp.float32) + bf1_ref[...]
    h1 = h1 * jax.nn.sigmoid(h1)
    h2 = jnp.dot(h1, wf2_ref[...], preferred_element_type=jnp.float32) + bf2_ref[...]
    h2 = h2 * jax.nn.sigmoid(h2)
    z = jnp.dot(h2, wf3_ref[...], preferred_element_type=jnp.float32) + bf3_ref[...]
    scale = jnp.maximum(z, 0.0) + jnp.log1p(jnp.exp(-jnp.abs(z)))
    coef = jnp.float32(4.0 * math.pi) * (jnp.float32(0.5) / vol_ref[0, 0])
    kern = coef / (safe_k * safe_k) * scale
    row = pl.program_id(0) * BLK_K + lax.broadcasted_iota(jnp.int32, (BLK_K, 1), 0)
    out_ref[...] = jnp.where(row == 0, 0.0, kern)


BLK_K = 4096


def _stage_c2(inv_cellT, volarr, Wf1, bf1r, Wf2, bf2r, Wf3, bf3r):
    return pl.pallas_call(
        _filter_body,
        grid=(MESH // BLK_K,),
        in_specs=[
            pl.BlockSpec((BLK_K, 3), lambda n: (n, 0)),
            pl.BlockSpec((3, 3), lambda n: (0, 0)),
            pl.BlockSpec(memory_space=pltpu.SMEM),
            pl.BlockSpec((3, FH), lambda n: (0, 0)),
            pl.BlockSpec((1, FH), lambda n: (0, 0)),
            pl.BlockSpec((FH, FH), lambda n: (0, 0)),
            pl.BlockSpec((1, FH), lambda n: (0, 0)),
            pl.BlockSpec((FH, 1), lambda n: (0, 0)),
            pl.BlockSpec((1, 1), lambda n: (0, 0)),
        ],
        out_specs=pl.BlockSpec((BLK_K, 1), lambda n: (n, 0)),
        out_shape=jax.ShapeDtypeStruct((MESH, 1), jnp.float32),
    )(KINT, inv_cellT, volarr, Wf1, bf1r, Wf2, bf2r, Wf3, bf3r)


def _dft_body(p_ref, kern_ref, f2r_ref, f2i_ref, bar_ref, bai_ref, phi_ref):
    X = jnp.sum(p_ref[...], axis=0)
    f2r = f2r_ref[...]
    f2i = f2i_ref[...]
    bar = bar_ref[...]
    bai = bai_ref[...]
    dot = functools.partial(jnp.dot, preferred_element_type=jnp.float32)
    r1 = dot(X, f2r)
    i1 = dot(X, f2i)
    r2 = dot(bar, r1) - dot(bai, i1)
    i2 = dot(bar, i1) + dot(bai, r1)
    k2 = kern_ref[...]
    kt = jnp.concatenate([k2, k2, k2, k2], axis=0)
    r3 = r2 * kt
    i3 = i2 * kt
    inv1 = jnp.float32(1.0 / (M * M))
    r4 = (dot(r3, f2r) + dot(i3, f2i)) * inv1
    i4 = (dot(i3, f2r) - dot(r3, f2i)) * inv1
    phi_ref[...] = (dot(bar, r4) + dot(bai, i4)) * jnp.float32(1.0 / M)


def _stage_c(partials, kern2):
    return pl.pallas_call(
        _dft_body,
        in_specs=[
            pl.BlockSpec((NCHUNK, C * M, M * M), lambda: (0, 0, 0)),
            pl.BlockSpec((M, M * M), lambda: (0, 0)),
            pl.BlockSpec((M * M, M * M), lambda: (0, 0)),
            pl.BlockSpec((M * M, M * M), lambda: (0, 0)),
            pl.BlockSpec((C * M, C * M), lambda: (0, 0)),
            pl.BlockSpec((C * M, C * M), lambda: (0, 0)),
        ],
        out_specs=pl.BlockSpec((C * M, M * M), lambda: (0, 0)),
        out_shape=jax.ShapeDtypeStruct((C * M, M * M), jnp.float32),
    )(partials, kern2, F2R, F2I, BAR, BAI)


def _gather_body(phi_hbm, srcT_hbm, idx8_hbm, w8_hbm, out_hbm,
                 phibuf, potbuf, ibuf0, ibuf1, wbuf0, wbuf1, srcbuf,
                 sem0, sem1, sems):
    wid = lax.axis_index("s") * 2 + lax.axis_index("c")
    p = wid // C
    c = wid % C
    base = p * CHUNK
    ibufs = (ibuf0, ibuf1)
    wbufs = (wbuf0, wbuf1)
    semsp = (sem0, sem1)
    cps = [None, None]

    def start(j):
        par = j & 1
        c1 = pltpu.make_async_copy(idx8_hbm.at[j, pl.ds(base, CHUNK)],
                                   ibufs[par], semsp[par])
        c2 = pltpu.make_async_copy(w8_hbm.at[j, pl.ds(base, CHUNK)],
                                   wbufs[par], semsp[par])
        c1.start()
        c2.start()
        cps[par] = (c1, c2)

    cp_phi = pltpu.make_async_copy(phi_hbm.at[c], phibuf, sems)
    cp_phi.start()
    cp_src = pltpu.make_async_copy(srcT_hbm.at[c, pl.ds(base, CHUNK)],
                                   srcbuf, sems)
    cp_src.start()
    start(0)

    def zero_body(g, _):
        for u in range(4):
            potbuf[pl.ds(g * 64 + u * 16, 16)] = jnp.zeros((16,), jnp.float32)
        return 0

    lax.fori_loop(0, G16 // 4, zero_body, 0)
    cp_phi.wait()
    cp_src.wait()

    for j in range(8):
        if j < 7:
            start(j + 1)
        par = j & 1
        ib = ibufs[par]
        wb = wbufs[par]
        for cp in cps[par]:
            cp.wait()

        def gat_body(g, _):
            for u in range(4):
                off = g * 64 + u * 16
                iv = ib[pl.ds(off, 16)]
                vals = plsc.load_gather(phibuf, [iv])
                potbuf[pl.ds(off, 16)] = (potbuf[pl.ds(off, 16)]
                                          + vals * wb[pl.ds(off, 16)])
            return 0

        lax.fori_loop(0, G16 // 4, gat_body, 0)

    def mul_body(g, _):
        for u in range(4):
            off = g * 64 + u * 16
            potbuf[pl.ds(off, 16)] = potbuf[pl.ds(off, 16)] * srcbuf[pl.ds(off, 16)]
        return 0

    lax.fori_loop(0, G16 // 4, mul_body, 0)
    pltpu.sync_copy(potbuf, out_hbm.at[c, pl.ds(base, CHUNK)])


def _stage_d(phi, srcT, idx8, w8):
    mesh = plsc.VectorSubcoreMesh(core_axis_name="c", subcore_axis_name="s",
                                  num_cores=2, num_subcores=16)
    k = pl.kernel(
        _gather_body,
        out_type=jax.ShapeDtypeStruct((C, NP), jnp.float32),
        mesh=mesh,
        scratch_types=[
            pltpu.VMEM((MESH,), jnp.float32),
            pltpu.VMEM((CHUNK,), jnp.float32),
            pltpu.VMEM((CHUNK,), jnp.int32),
            pltpu.VMEM((CHUNK,), jnp.int32),
            pltpu.VMEM((CHUNK,), jnp.float32),
            pltpu.VMEM((CHUNK,), jnp.float32),
            pltpu.VMEM((CHUNK,), jnp.float32),
            pltpu.SemaphoreType.DMA,
            pltpu.SemaphoreType.DMA,
            pltpu.SemaphoreType.DMA,
        ],
        compiler_params=pltpu.CompilerParams(
            needs_layout_passes=False, use_tc_tiling_on_sc=False),
    )
    return k(phi, srcT, idx8, w8)


def _reduce_body(e_ref, out_ref):
    out_ref[...] = jnp.sum(e_ref[...], axis=0)[:N]


def _stage_e(epart):
    return pl.pallas_call(
        _reduce_body,
        in_specs=[pl.BlockSpec((C, NP), lambda: (0, 0))],
        out_specs=pl.BlockSpec((N,), lambda: (0,)),
        out_shape=jax.ShapeDtypeStruct((N,), jnp.float32),
    )(epart)


def kernel(invariant_features, positions, cell, W1, b1, W2, b2,
           Wf1, bf1, Wf2, bf2, Wf3, bf3):
    inv_cell = jnp.linalg.inv(cell)
    inv_cellT = inv_cell.T
    vol = jnp.abs(jnp.linalg.det(cell))
    volarr = vol.reshape(1, 1)
    posT = positions.T
    b1c = b1.reshape(H, 1)
    b2c = b2.reshape(C, 1)
    bf1r = bf1.reshape(1, FH)
    bf2r = bf2.reshape(1, FH)
    bf3r = bf3.reshape(1, 1)

    srcT, idx8, w8 = _stage_a(invariant_features, posT, inv_cellT, W1, b1c, W2, b2c)
    partials = _stage_b(srcT, idx8, w8)
    kern_col = _stage_c2(inv_cellT, volarr, Wf1, bf1r, Wf2, bf2r, Wf3, bf3r)
    kern2 = kern_col.reshape(M, M * M)
    phi = _stage_c(partials.reshape(NCHUNK, C * M, M * M), kern2)
    epart = _stage_d(phi.reshape(C, MESH), srcT, idx8, w8)
    return _stage_e(epart)
